# baseline (speedup 1.0000x reference)
import functools

import jax
import jax.numpy as jnp
from jax import lax
from jax.experimental import pallas as pl
from jax.experimental.pallas import tpu as pltpu
from jax.experimental.pallas import tpu_sc as plsc

N = 50000
E = 800000
D = 64
H = 32
SUB = 128
NSUB = E // SUB
NT = 16
NP = 50048
ROWS_PER_TILE = NP // NT
ZROWS = 136
ZITER = 23
BR = 2000


def _proj_body(feats, nt, w0, b0, wa, ba, out):
    p = jnp.dot(feats[...], w0[...], preferred_element_type=jnp.float32)
    p = p + b0[...]
    p = jnp.where(nt[...] == 0, p, 0.0)
    x = jnp.dot(p, wa[...], preferred_element_type=jnp.float32) + ba[...]
    out[0] = x[:, :H]
    out[1] = x[:, H:]


def _project(feats, node_types, w0, b0, wa, ba):
    grid = N // BR
    return pl.pallas_call(
        _proj_body,
        grid=(grid,),
        in_specs=[
            pl.BlockSpec((BR, D), lambda i: (i, 0)),
            pl.BlockSpec((BR, 1), lambda i: (i, 0)),
            pl.BlockSpec((D, D), lambda i: (0, 0)),
            pl.BlockSpec((1, D), lambda i: (0, 0)),
            pl.BlockSpec((D, D), lambda i: (0, 0)),
            pl.BlockSpec((1, D), lambda i: (0, 0)),
        ],
        out_specs=pl.BlockSpec((2, BR, H), lambda i: (0, i, 0)),
        out_shape=jax.ShapeDtypeStruct((2, NP, H), jnp.float32),
    )(feats, node_types.reshape(N, 1), w0, b0.reshape(1, D),
      wa, ba.reshape(1, D))


def _ln_gelu_body(y, out):
    v = jnp.concatenate([y[0], y[1]], axis=-1)
    mu = jnp.mean(v, axis=-1, keepdims=True)
    var = jnp.mean((v - mu) ** 2, axis=-1, keepdims=True)
    vn = (v - mu) / jnp.sqrt(var + 1e-5)
    out[...] = 0.5 * vn * (1.0 + lax.erf(vn * (2.0 ** -0.5)))


def _ln_gelu(y2):
    grid = N // BR
    return pl.pallas_call(
        _ln_gelu_body,
        grid=(grid,),
        in_specs=[pl.BlockSpec((2, BR, H), lambda i: (0, i, 0))],
        out_specs=pl.BlockSpec((BR, D), lambda i: (i, 0)),
        out_shape=jax.ShapeDtypeStruct((N, D), jnp.float32),
    )(y2)


_GDN = lax.GatherDimensionNumbers(
    offset_dims=(), collapsed_slice_dims=(0,), start_index_map=(0,))


def _splat(vv, e):
    idx = jnp.full((16, 1), e, jnp.int32)
    return lax.gather(vv, idx, _GDN, (1,),
                      mode=lax.GatherScatterMode.PROMISE_IN_BOUNDS)


def _sc_body(x2, ei1, v1, ei2, v2, ei3, v3, scales, u2, y2,
             acc, dstb, srcb, valb, rbig, zbuf,
             svecb, stsem, gsem, scsem):
    c = lax.axis_index("c")
    t = lax.axis_index("s")
    xoff = c * NP

    zero16 = jnp.zeros((16,), jnp.float32)

    def zrow(i, _):
        zbuf[i, pl.ds(0, 16)] = zero16
        zbuf[i, pl.ds(16, 16)] = zero16
        return 0
    lax.fori_loop(0, ZROWS, zrow, 0)

    def zcp(i, _):
        pltpu.sync_copy(zbuf, acc.at[pl.ds(t * ROWS_PER_TILE + i * ZROWS,
                                           ZROWS)])
        return 0
    lax.fori_loop(0, ZITER, zcp, 0)

    pltpu.sync_copy(scales, svecb)
    sv = svecb[...]
    s2 = _splat(sv, 0)
    s3 = _splat(sv, 1)

    NB = NSUB // NT
    SG = 26
    NG = NB // SG
    TAIL = NSUB - NT * NB

    def scale_rows(j, b0, scale):
        def grp(g, _):
            vv = valb[j, pl.ds(g * 16, 16)]
            if scale is not None:
                vv = vv * scale
            for e in range(16):
                sp = _splat(vv, e)
                rr = b0 + g * 16 + e
                rbig[rr, pl.ds(0, 16)] = rbig[rr, pl.ds(0, 16)] * sp
                rbig[rr, pl.ds(16, 16)] = rbig[rr, pl.ds(16, 16)] * sp
            return 0
        lax.fori_loop(0, 8, grp, 0)

    def spmm(ei, vals, table, scale):
        tbl = table.at[pl.ds(xoff, NP)]

        def fire(j):
            b0 = lax.rem(j, 4) * SUB
            pltpu.async_copy(tbl.at[srcb.at[j]],
                             rbig.at[pl.ds(b0, SUB)], gsem)

        def drain_g():
            pltpu.make_async_copy(tbl.at[srcb.at[0]],
                                  rbig.at[pl.ds(0, SUB)], gsem).wait()

        def drain_s():
            pltpu.make_async_copy(rbig.at[pl.ds(0, SUB)],
                                  acc.at[dstb.at[0]], scsem).wait()

        def group(gi, _):
            s0 = t * NB + gi * SG
            d1 = pltpu.async_copy(ei.at[0].at[pl.ds(s0, SG)], dstb, stsem)
            d2 = pltpu.async_copy(ei.at[1].at[pl.ds(s0, SG)], srcb, stsem)
            d3 = pltpu.async_copy(vals.at[pl.ds(s0, SG)], valb, stsem)
            d1.wait()
            d2.wait()
            d3.wait()
            for j in range(3):
                fire(jnp.int32(j))

            def slot(j, _):
                @pl.when(j + 3 < SG)
                def _ahead():
                    @pl.when(j >= 1)
                    def _ds():
                        drain_s()
                    fire(j + 3)
                drain_g()
                b0 = lax.rem(j, 4) * SUB
                scale_rows(j, b0, scale)
                pltpu.async_copy(rbig.at[pl.ds(b0, SUB)],
                                 acc.at[dstb.at[j]], scsem, add=True)
                return 0
            lax.fori_loop(0, SG, slot, 0)
            for _ in range(4):
                drain_s()
            return 0
        lax.fori_loop(0, NG, group, 0)

        @pl.when(t < TAIL)
        def _tail():
            s = NT * NB + t
            pltpu.sync_copy(ei.at[0].at[pl.ds(s, 1)], dstb.at[pl.ds(0, 1)])
            pltpu.sync_copy(ei.at[1].at[pl.ds(s, 1)], srcb.at[pl.ds(0, 1)])
            pltpu.sync_copy(vals.at[pl.ds(s, 1)], valb.at[pl.ds(0, 1)])
            pltpu.async_copy(tbl.at[srcb.at[0]],
                             rbig.at[pl.ds(0, SUB)], gsem).wait()
            scale_rows(jnp.int32(0), 0, scale)
            pltpu.sync_copy(rbig.at[pl.ds(0, SUB)], acc.at[dstb.at[0]],
                            add=True)

    plsc.subcore_barrier()
    spmm(ei1, v1, x2, None)
    plsc.subcore_barrier()

    def ucp(i, _):
        r0 = t * ROWS_PER_TILE + i * ZROWS
        pltpu.sync_copy(acc.at[pl.ds(r0, ZROWS)],
                        u2.at[pl.ds(xoff + r0, ZROWS)])
        pltpu.sync_copy(zbuf, acc.at[pl.ds(r0, ZROWS)])
        return 0
    lax.fori_loop(0, ZITER, ucp, 0)
    plsc.subcore_barrier()

    spmm(ei2, v2, u2, s2)
    spmm(ei3, v3, x2, s3)
    plsc.subcore_barrier()

    def ocp(i, _):
        r0 = t * ROWS_PER_TILE + i * ZROWS
        pltpu.sync_copy(acc.at[pl.ds(r0, ZROWS)],
                        y2.at[pl.ds(xoff + r0, ZROWS)])
        return 0
    lax.fori_loop(0, ZITER, ocp, 0)


def _sc_spmms(x2, ei1, v1, ei2, v2, ei3, v3, scales):
    mesh = plsc.VectorSubcoreMesh(core_axis_name="c", subcore_axis_name="s")
    f = pl.kernel(
        _sc_body,
        out_type=(jax.ShapeDtypeStruct((2 * NP, H), jnp.float32),
                  jax.ShapeDtypeStruct((2 * NP, H), jnp.float32)),
        mesh=mesh,
        scratch_types=[
            pltpu.VMEM_SHARED((NP, H), jnp.float32),
            pltpu.VMEM((26, SUB), jnp.int32),
            pltpu.VMEM((26, SUB), jnp.int32),
            pltpu.VMEM((26, SUB), jnp.float32),
            pltpu.VMEM((4 * SUB, H), jnp.float32),
            pltpu.VMEM((ZROWS, H), jnp.float32),
            pltpu.VMEM((16,), jnp.float32),
            pltpu.SemaphoreType.DMA,
            pltpu.SemaphoreType.DMA,
            pltpu.SemaphoreType.DMA,
        ],
        compiler_params=pltpu.CompilerParams(use_tc_tiling_on_sc=False),
    )
    return f(x2, ei1, v1, ei2, v2, ei3, v3, scales)


def _pick(i, arrs):
    return lax.switch(i, [lambda a=a: a for a in arrs])


def kernel(node_feats_0, node_types, adj0_edge_index, adj0_values,
           adj1_edge_index, adj1_values, adj2_edge_index, adj2_values,
           adj3_edge_index, adj3_values, idx_seq0, idx_seq_last,
           idx_res_last, W0_w, W0_b, aff_w, aff_b, as_seq, as_last_seq,
           as_last_res, attn1_w, attn1_b, attn2_w, attn2_b):
    x2 = _project(node_feats_0, node_types, W0_w, W0_b, aff_w, aff_b)
    x2 = x2.reshape(2 * NP, H)

    i0 = idx_seq0[0]
    il = idx_seq_last[0]
    ir = idx_res_last[0]
    w1 = jax.nn.softmax(as_seq, axis=-1)[0, i0]
    w2 = jax.nn.softmax(as_last_seq, axis=-1)[il]
    w3 = jax.nn.softmax(as_last_res, axis=-1)[0, ir]
    scales = jnp.zeros((16,), jnp.float32).at[0].set(w1 * w2).at[1].set(w3)

    es = (adj0_edge_index, adj1_edge_index, adj2_edge_index, adj3_edge_index)
    vs = (adj0_values, adj1_values, adj2_values, adj3_values)
    ei1 = _pick(i0, es[:3]).reshape(2, NSUB, SUB)
    v1 = _pick(i0, vs[:3]).reshape(NSUB, SUB)
    ei2 = _pick(il, (es[0], es[2])).reshape(2, NSUB, SUB)
    v2 = _pick(il, (vs[0], vs[2])).reshape(NSUB, SUB)
    ei3 = _pick(ir, (es[0], es[2], es[3])).reshape(2, NSUB, SUB)
    v3 = _pick(ir, (vs[0], vs[2], vs[3])).reshape(NSUB, SUB)

    _u2, y2 = _sc_spmms(x2, ei1, v1, ei2, v2, ei3, v3, scales)
    return _ln_gelu(y2.reshape(2, NP, H))
